# baseline (speedup 1.0000x reference)
import functools
import math

import jax
import jax.numpy as jnp
import numpy as np
from jax import lax
from jax.experimental import pallas as pl
from jax.experimental.pallas import tpu as pltpu
from jax.experimental.pallas import tpu_sc as plsc

_C = 256


def _pack_pairs_f32(lo, hi):
    lo16 = jnp.asarray(lo, jnp.bfloat16).view(jnp.uint16).astype(jnp.uint32)
    hi16 = jnp.asarray(hi, jnp.bfloat16).view(jnp.uint16).astype(jnp.uint32)
    return (lo16 | (hi16 << 16)).astype(jnp.int32)


def _make_sc_call(n_tokens, d):
    nc, ns = 2, 16
    nw = nc * ns
    n_per_w = n_tokens // nw
    chunks = n_per_w // _C
    half = d // 2
    hw = half // 16

    def body(e_hbm, a_hbm, v_hbm, t_hbm, we_hbm, wa_hbm, wv_hbm, qt1_hbm,
             rt_hbm, out_hbm, we_v, wa_v, wv_v, qt1_v, rt_v, wea_v, ei_v,
             ai_v, vi_v, ti_v, o0_v, o1_v, sem0, sem1):
        wid = lax.axis_index("s") * nc + lax.axis_index("c")

        pltpu.sync_copy(we_hbm, we_v)
        pltpu.sync_copy(wa_hbm, wa_v)
        pltpu.sync_copy(wv_hbm, wv_v)
        pltpu.sync_copy(qt1_hbm, qt1_v)
        pltpu.sync_copy(rt_hbm, rt_v)

        lanes = lax.iota(jnp.int32, 16)
        segs = [lanes + 16 * k for k in range(hw)]

        def build_ea(ea, carry):
            web = lax.shift_right_logical(ea, 4) * half
            wab = lax.bitwise_and(ea, 15) * half
            ob = ea * half
            for k in range(hw):
                we = plsc.bitcast(we_v[pl.ds(web + 16 * k, 16)], jnp.bfloat16)
                wa = plsc.bitcast(wa_v[pl.ds(wab + 16 * k, 16)], jnp.bfloat16)
                wea_v[pl.ds(ob + 16 * k, 16)] = plsc.bitcast(we + wa,
                                                             jnp.int32)
            return carry

        lax.fori_loop(0, 512, build_ea, 0)

        hc = _C // 2

        def chunk_body(ci, carry):
            base = wid * n_per_w + ci * _C
            pltpu.sync_copy(e_hbm.at[pl.ds(base, _C)], ei_v)
            pltpu.sync_copy(a_hbm.at[pl.ds(base, _C)], ai_v)
            pltpu.sync_copy(v_hbm.at[pl.ds(base, _C)], vi_v)
            pltpu.sync_copy(t_hbm.at[pl.ds(base, _C)], ti_v)

            def half_body(h, buf, sem):
                @pl.when(ci > 0)
                def _():
                    pltpu.make_async_copy(
                        out_hbm.at[pl.ds(0, hc * d)], buf, sem).wait()

                def group_body(g):
                    off = h * hc + g * 16
                    e = ei_v[pl.ds(off, 16)]
                    a = ai_v[pl.ds(off, 16)]
                    v = vi_v[pl.ds(off, 16)]
                    t = ti_v[pl.ds(off, 16)]
                    eab = (e * 16 + a) * half
                    vb = v * half
                    qb = lax.shift_right_logical(t, 6) * half
                    rb = lax.bitwise_and(t, 63) * half

                    def tok_body(l):
                        idx = jnp.full((16,), 0, jnp.int32) + l

                        def splat(x):
                            return jnp.take_along_axis(
                                x, idx, axis=0,
                                mode="promise_in_bounds") + lanes

                        eabs = splat(eab)
                        vbs = splat(vb)
                        qbs = splat(qb)
                        rbs = splat(rb)
                        obase = (g * 16 + l) * d

                        for k in range(hw):
                            o = 16 * k

                            def bf(tab, bs):
                                w = plsc.load_gather(
                                    tab.at[pl.ds(o, tab.shape[0] - o)], [bs])
                                return plsc.bitcast(w, jnp.bfloat16)

                            def unpk(x):
                                return plsc.unpack(
                                    x, format=plsc.PackFormat.INTERLEAVED,
                                    preferred_element_type=jnp.float32)

                            w0, w1 = unpk(bf(wea_v, eabs) + bf(wv_v, vbs))
                            s1, c1 = unpk(bf(qt1_v, qbs))
                            s2, c2 = unpk(bf(rt_v, rbs))
                            sin16 = w0 + s1 * c2 + c1 * s2
                            cos16 = w1 + (c1 * c2 - s1 * s2)
                            buf[pl.ds(obase + 16 * k, 16)] = sin16
                            buf[pl.ds(obase + half + 16 * k, 16)] = cos16

                    plsc.parallel_loop(0, 16, unroll=8)(tok_body)

                plsc.parallel_loop(0, hc // 16)(group_body)
                pltpu.async_copy(
                    buf, out_hbm.at[pl.ds((base + h * hc) * d, hc * d)], sem)

            half_body(0, o0_v, sem0)
            half_body(1, o1_v, sem1)
            return carry

        lax.fori_loop(0, chunks, chunk_body, 0)
        pltpu.make_async_copy(out_hbm.at[pl.ds(0, hc * d)], o0_v, sem0).wait()
        pltpu.make_async_copy(out_hbm.at[pl.ds(0, hc * d)], o1_v, sem1).wait()

    mesh = plsc.VectorSubcoreMesh(
        core_axis_name="c", subcore_axis_name="s",
        num_cores=nc, num_subcores=ns)
    return pl.kernel(
        body,
        out_type=jax.ShapeDtypeStruct((n_tokens * d,), jnp.float32),
        mesh=mesh,
        compiler_params=pltpu.CompilerParams(needs_layout_passes=False),
        scratch_types=[
            pltpu.VMEM((32 * 64,), jnp.int32),
            pltpu.VMEM((16 * 64,), jnp.int32),
            pltpu.VMEM((32 * 64,), jnp.int32),
            pltpu.VMEM((64 * 64,), jnp.int32),
            pltpu.VMEM((64 * 64,), jnp.int32),
            pltpu.VMEM((512 * 64,), jnp.int32),
            pltpu.VMEM((_C,), jnp.int32),
            pltpu.VMEM((_C,), jnp.int32),
            pltpu.VMEM((_C,), jnp.int32),
            pltpu.VMEM((_C,), jnp.int32),
            pltpu.VMEM((_C // 2 * 128,), jnp.float32),
            pltpu.VMEM((_C // 2 * 128,), jnp.float32),
            pltpu.SemaphoreType.DMA,
            pltpu.SemaphoreType.DMA,
        ],
    )


def kernel(entity, attribute, value_binned, time, W_entity, W_attribute, W_value_binned):
    B, S = entity.shape
    D = W_entity.shape[1]
    half = D // 2
    N = B * S

    ratio = math.log(10000.0) / half
    f = np.exp(-ratio * np.arange(half, dtype=np.float64))
    qa = (64.0 * np.arange(64, dtype=np.float64))[:, None] * f[None, :]
    ra = np.arange(64, dtype=np.float64)[:, None] * f[None, :]
    s1, c1 = np.sin(qa), np.cos(qa)
    s2, c2 = np.sin(ra), np.cos(ra)
    qt1 = _pack_pairs_f32(s1, c1).reshape(-1)
    rt = _pack_pairs_f32(s2, c2).reshape(-1)

    def packw(w):
        return _pack_pairs_f32(w[:, :half], w[:, half:]).reshape(-1)

    call = _make_sc_call(N, D)
    out = call(
        entity.reshape(-1), attribute.reshape(-1), value_binned.reshape(-1),
        time.reshape(-1), packw(W_entity), packw(W_attribute),
        packw(W_value_binned), qt1, rt)
    return out.reshape(B, S, D)


# speedup vs baseline: 1.2964x; 1.2964x over previous
import functools
import math

import jax
import jax.numpy as jnp
import numpy as np
from jax import lax
from jax.experimental import pallas as pl
from jax.experimental.pallas import tpu as pltpu
from jax.experimental.pallas import tpu_sc as plsc

_C = 256


def _pack_pairs_f32(lo, hi):
    lo16 = jnp.asarray(lo, jnp.bfloat16).view(jnp.uint16).astype(jnp.uint32)
    hi16 = jnp.asarray(hi, jnp.bfloat16).view(jnp.uint16).astype(jnp.uint32)
    return (lo16 | (hi16 << 16)).astype(jnp.int32)


def _make_sc_call(n_tokens, d):
    nc, ns = 2, 16
    nw = nc * ns
    n_per_w = n_tokens // nw
    chunks = n_per_w // _C
    half = d // 2
    hw = half // 16

    def body(idx_hbm, we_hbm, wa_hbm, wv_hbm, qt1_hbm,
             rt_hbm, out_hbm, we_v, wa_v, wv_v, qt1_v, rt_v, wea_v, idx_v,
             o0_v, o1_v, sem0, sem1):
        wid = lax.axis_index("s") * nc + lax.axis_index("c")

        pltpu.sync_copy(we_hbm, we_v)
        pltpu.sync_copy(wa_hbm, wa_v)
        pltpu.sync_copy(wv_hbm, wv_v)
        pltpu.sync_copy(qt1_hbm, qt1_v)
        pltpu.sync_copy(rt_hbm, rt_v)

        lanes = lax.iota(jnp.int32, 16)
        segs = [lanes + 16 * k for k in range(hw)]

        def build_ea(ea, carry):
            web = lax.shift_right_logical(ea, 4) * half
            wab = lax.bitwise_and(ea, 15) * half
            ob = ea * half
            for k in range(hw):
                we = plsc.bitcast(we_v[pl.ds(web + 16 * k, 16)], jnp.bfloat16)
                wa = plsc.bitcast(wa_v[pl.ds(wab + 16 * k, 16)], jnp.bfloat16)
                wea_v[pl.ds(ob + 16 * k, 16)] = plsc.bitcast(we + wa,
                                                             jnp.int32)
            return carry

        lax.fori_loop(0, 512, build_ea, 0)

        hc = _C // 2

        def chunk_body(ci, carry):
            base = wid * n_per_w + ci * _C
            pltpu.sync_copy(idx_hbm.at[:, pl.ds(base, _C)], idx_v)

            def half_body(h, buf, sem):
                @pl.when(ci > 0)
                def _():
                    pltpu.make_async_copy(
                        out_hbm.at[pl.ds(0, hc * d)], buf, sem).wait()

                def group_body(g):
                    off = h * hc + g * 16
                    e = idx_v[0, pl.ds(off, 16)]
                    a = idx_v[1, pl.ds(off, 16)]
                    v = idx_v[2, pl.ds(off, 16)]
                    t = idx_v[3, pl.ds(off, 16)]
                    eab = (e * 16 + a) * half
                    vb = v * half
                    qb = lax.shift_right_logical(t, 6) * half
                    rb = lax.bitwise_and(t, 63) * half

                    def tok_body(l):
                        idx = jnp.full((16,), 0, jnp.int32) + l

                        def splat(x):
                            return jnp.take_along_axis(
                                x, idx, axis=0,
                                mode="promise_in_bounds") + lanes

                        eabs = splat(eab)
                        vbs = splat(vb)
                        qbs = splat(qb)
                        rbs = splat(rb)
                        obase = (g * 16 + l) * d

                        for k in range(hw):
                            o = 16 * k

                            def bf(tab, bs):
                                w = plsc.load_gather(
                                    tab.at[pl.ds(o, tab.shape[0] - o)], [bs])
                                return plsc.bitcast(w, jnp.bfloat16)

                            def unpk(x):
                                return plsc.unpack(
                                    x, format=plsc.PackFormat.INTERLEAVED,
                                    preferred_element_type=jnp.float32)

                            w0, w1 = unpk(bf(wea_v, eabs) + bf(wv_v, vbs))
                            s1, c1 = unpk(bf(qt1_v, qbs))
                            s2, c2 = unpk(bf(rt_v, rbs))
                            sin16 = w0 + s1 * c2 + c1 * s2
                            cos16 = w1 + (c1 * c2 - s1 * s2)
                            buf[pl.ds(obase + 16 * k, 16)] = sin16
                            buf[pl.ds(obase + half + 16 * k, 16)] = cos16

                    plsc.parallel_loop(0, 16, unroll=4)(tok_body)

                plsc.parallel_loop(0, hc // 16)(group_body)
                pltpu.async_copy(
                    buf, out_hbm.at[pl.ds((base + h * hc) * d, hc * d)], sem)

            half_body(0, o0_v, sem0)
            half_body(1, o1_v, sem1)
            return carry

        lax.fori_loop(0, chunks, chunk_body, 0)
        pltpu.make_async_copy(out_hbm.at[pl.ds(0, hc * d)], o0_v, sem0).wait()
        pltpu.make_async_copy(out_hbm.at[pl.ds(0, hc * d)], o1_v, sem1).wait()

    mesh = plsc.VectorSubcoreMesh(
        core_axis_name="c", subcore_axis_name="s",
        num_cores=nc, num_subcores=ns)
    return pl.kernel(
        body,
        out_type=jax.ShapeDtypeStruct((n_tokens * d,), jnp.float32),
        mesh=mesh,
        compiler_params=pltpu.CompilerParams(needs_layout_passes=False),
        scratch_types=[
            pltpu.VMEM((32 * 64,), jnp.int32),
            pltpu.VMEM((16 * 64,), jnp.int32),
            pltpu.VMEM((32 * 64,), jnp.int32),
            pltpu.VMEM((64 * 64,), jnp.int32),
            pltpu.VMEM((64 * 64,), jnp.int32),
            pltpu.VMEM((512 * 64,), jnp.int32),
            pltpu.VMEM((4, _C), jnp.int32),
            pltpu.VMEM((_C // 2 * 128,), jnp.float32),
            pltpu.VMEM((_C // 2 * 128,), jnp.float32),
            pltpu.SemaphoreType.DMA,
            pltpu.SemaphoreType.DMA,
        ],
    )


def kernel(entity, attribute, value_binned, time, W_entity, W_attribute, W_value_binned):
    B, S = entity.shape
    D = W_entity.shape[1]
    half = D // 2
    N = B * S

    ratio = math.log(10000.0) / half
    f = np.exp(-ratio * np.arange(half, dtype=np.float64))
    qa = (64.0 * np.arange(64, dtype=np.float64))[:, None] * f[None, :]
    ra = np.arange(64, dtype=np.float64)[:, None] * f[None, :]
    s1, c1 = np.sin(qa), np.cos(qa)
    s2, c2 = np.sin(ra), np.cos(ra)
    qt1 = _pack_pairs_f32(s1, c1).reshape(-1)
    rt = _pack_pairs_f32(s2, c2).reshape(-1)

    def packw(w):
        return _pack_pairs_f32(w[:, :half], w[:, half:]).reshape(-1)

    idx4 = jnp.stack([entity.reshape(-1), attribute.reshape(-1),
                      value_binned.reshape(-1), time.reshape(-1)])

    call = _make_sc_call(N, D)
    out = call(
        idx4, packw(W_entity), packw(W_attribute),
        packw(W_value_binned), qt1, rt)
    return out.reshape(B, S, D)


# speedup vs baseline: 1.4452x; 1.1147x over previous
import functools
import math

import jax
import jax.numpy as jnp
import numpy as np
from jax import lax
from jax.experimental import pallas as pl
from jax.experimental.pallas import tpu as pltpu
from jax.experimental.pallas import tpu_sc as plsc

_C = 256


def _pack_pairs_f32(lo, hi):
    lo16 = jnp.asarray(lo, jnp.bfloat16).view(jnp.uint16).astype(jnp.uint32)
    hi16 = jnp.asarray(hi, jnp.bfloat16).view(jnp.uint16).astype(jnp.uint32)
    return (lo16 | (hi16 << 16)).astype(jnp.int32)


def _make_sc_call(n_tokens, d):
    nc, ns = 2, 16
    nw = nc * ns
    n_per_w = n_tokens // nw
    chunks = n_per_w // _C
    half = d // 2
    hw = half // 16

    def body(idx_hbm, we_hbm, wa_hbm, wv_hbm, qt1_hbm,
             rt_hbm, out_hbm, we_v, wa_v, wv_v, qt1_v, rt_v, wea_v, ixa_v,
             ixb_v, o0_v, o1_v, sem0, sem1, isema, isemb):
        wid = lax.axis_index("s") * nc + lax.axis_index("c")

        pltpu.sync_copy(we_hbm, we_v)
        pltpu.sync_copy(wa_hbm, wa_v)
        pltpu.sync_copy(wv_hbm, wv_v)
        pltpu.sync_copy(qt1_hbm, qt1_v)
        pltpu.sync_copy(rt_hbm, rt_v)

        lanes = lax.iota(jnp.int32, 16)
        segs = [lanes + 16 * k for k in range(hw)]

        def build_ea(ea, carry):
            web = lax.shift_right_logical(ea, 4) * half
            wab = lax.bitwise_and(ea, 15) * half
            ob = ea * half
            for k in range(hw):
                we = plsc.bitcast(we_v[pl.ds(web + 16 * k, 16)], jnp.bfloat16)
                wa = plsc.bitcast(wa_v[pl.ds(wab + 16 * k, 16)], jnp.bfloat16)
                wea_v[pl.ds(ob + 16 * k, 16)] = plsc.bitcast(we + wa,
                                                             jnp.int32)
            return carry

        lax.fori_loop(0, 512, build_ea, 0)

        hc = _C // 2
        wbase = wid * n_per_w

        def compute_chunk(base, idx_v, not_first):
            def half_body(h, buf, sem):
                def drain():
                    pltpu.make_async_copy(
                        out_hbm.at[pl.ds(0, hc * d)], buf, sem).wait()

                if not_first is True:
                    drain()
                else:
                    pl.when(not_first)(drain)

                def group_body(g):
                    off = h * hc + g * 16
                    e = idx_v[0, pl.ds(off, 16)]
                    a = idx_v[1, pl.ds(off, 16)]
                    v = idx_v[2, pl.ds(off, 16)]
                    t = idx_v[3, pl.ds(off, 16)]
                    eab = (e * 16 + a) * half
                    vb = v * half
                    qb = lax.shift_right_logical(t, 6) * half
                    rb = lax.bitwise_and(t, 63) * half

                    def tok_body(l):
                        idx = jnp.full((16,), 0, jnp.int32) + l

                        def splat(x):
                            return jnp.take_along_axis(
                                x, idx, axis=0,
                                mode="promise_in_bounds") + lanes

                        eabs = splat(eab)
                        vbs = splat(vb)
                        qbs = splat(qb)
                        rbs = splat(rb)
                        obase = (g * 16 + l) * d

                        for k in range(hw):
                            o = 16 * k

                            def bf(tab, bs):
                                w = plsc.load_gather(
                                    tab.at[pl.ds(o, tab.shape[0] - o)], [bs])
                                return plsc.bitcast(w, jnp.bfloat16)

                            def unpk(x):
                                return plsc.unpack(
                                    x, format=plsc.PackFormat.INTERLEAVED,
                                    preferred_element_type=jnp.float32)

                            w0, w1 = unpk(bf(wea_v, eabs) + bf(wv_v, vbs))
                            s1, c1 = unpk(bf(qt1_v, qbs))
                            s2, c2 = unpk(bf(rt_v, rbs))
                            sin16 = w0 + s1 * c2 + c1 * s2
                            cos16 = w1 + (c1 * c2 - s1 * s2)
                            buf[pl.ds(obase + 16 * k, 16)] = sin16
                            buf[pl.ds(obase + half + 16 * k, 16)] = cos16

                    plsc.parallel_loop(0, 16, unroll=4)(tok_body)

                plsc.parallel_loop(0, hc // 16)(group_body)
                pltpu.async_copy(
                    buf, out_hbm.at[pl.ds((base + h * hc) * d, hc * d)], sem)

            half_body(0, o0_v, sem0)
            half_body(1, o1_v, sem1)

        def idx_fetch(ci, buf, sem):
            pltpu.async_copy(
                idx_hbm.at[:, pl.ds(wbase + ci * _C, _C)], buf, sem)

        def idx_wait(buf, sem):
            pltpu.make_async_copy(
                idx_hbm.at[:, pl.ds(0, _C)], buf, sem).wait()

        idx_fetch(0, ixa_v, isema)

        def pair_body(p, carry):
            c0 = p * 2
            idx_wait(ixa_v, isema)
            idx_fetch(c0 + 1, ixb_v, isemb)
            compute_chunk(wbase + c0 * _C, ixa_v, c0 > 0)
            idx_wait(ixb_v, isemb)
            idx_fetch(c0 + 2, ixa_v, isema)
            compute_chunk(wbase + (c0 + 1) * _C, ixb_v, True)
            return carry

        lax.fori_loop(0, chunks // 2, pair_body, 0)
        idx_wait(ixa_v, isema)
        compute_chunk(wbase + (chunks - 1) * _C, ixa_v, True)
        pltpu.make_async_copy(out_hbm.at[pl.ds(0, hc * d)], o0_v, sem0).wait()
        pltpu.make_async_copy(out_hbm.at[pl.ds(0, hc * d)], o1_v, sem1).wait()

    mesh = plsc.VectorSubcoreMesh(
        core_axis_name="c", subcore_axis_name="s",
        num_cores=nc, num_subcores=ns)
    return pl.kernel(
        body,
        out_type=jax.ShapeDtypeStruct((n_tokens * d,), jnp.float32),
        mesh=mesh,
        compiler_params=pltpu.CompilerParams(needs_layout_passes=False),
        scratch_types=[
            pltpu.VMEM((32 * 64,), jnp.int32),
            pltpu.VMEM((16 * 64,), jnp.int32),
            pltpu.VMEM((32 * 64,), jnp.int32),
            pltpu.VMEM((64 * 64,), jnp.int32),
            pltpu.VMEM((64 * 64,), jnp.int32),
            pltpu.VMEM((512 * 64,), jnp.int32),
            pltpu.VMEM((4, _C), jnp.int32),
            pltpu.VMEM((4, _C), jnp.int32),
            pltpu.VMEM((_C // 2 * 128,), jnp.float32),
            pltpu.VMEM((_C // 2 * 128,), jnp.float32),
            pltpu.SemaphoreType.DMA,
            pltpu.SemaphoreType.DMA,
            pltpu.SemaphoreType.DMA,
            pltpu.SemaphoreType.DMA,
        ],
    )


def kernel(entity, attribute, value_binned, time, W_entity, W_attribute, W_value_binned):
    B, S = entity.shape
    D = W_entity.shape[1]
    half = D // 2
    N = B * S

    ratio = math.log(10000.0) / half
    f = np.exp(-ratio * np.arange(half, dtype=np.float64))
    qa = (64.0 * np.arange(64, dtype=np.float64))[:, None] * f[None, :]
    ra = np.arange(64, dtype=np.float64)[:, None] * f[None, :]
    s1, c1 = np.sin(qa), np.cos(qa)
    s2, c2 = np.sin(ra), np.cos(ra)
    qt1 = _pack_pairs_f32(s1, c1).reshape(-1)
    rt = _pack_pairs_f32(s2, c2).reshape(-1)

    def packw(w):
        return _pack_pairs_f32(w[:, :half], w[:, half:]).reshape(-1)

    idx4 = jnp.stack([entity.reshape(-1), attribute.reshape(-1),
                      value_binned.reshape(-1), time.reshape(-1)])

    call = _make_sc_call(N, D)
    out = call(
        idx4, packw(W_entity), packw(W_attribute),
        packw(W_value_binned), qt1, rt)
    return out.reshape(B, S, D)


# speedup vs baseline: 1.5269x; 1.0566x over previous
import functools
import math

import jax
import jax.numpy as jnp
import numpy as np
from jax import lax
from jax.experimental import pallas as pl
from jax.experimental.pallas import tpu as pltpu
from jax.experimental.pallas import tpu_sc as plsc

_C = 256


def _pack_pairs_f32(lo, hi):
    lo16 = jnp.asarray(lo, jnp.bfloat16).view(jnp.uint16).astype(jnp.uint32)
    hi16 = jnp.asarray(hi, jnp.bfloat16).view(jnp.uint16).astype(jnp.uint32)
    return (lo16 | (hi16 << 16)).astype(jnp.int32)


def _make_sc_call(n_tokens, d):
    nc, ns = 2, 16
    nw = nc * ns
    n_per_w = n_tokens // nw
    chunks = n_per_w // _C
    half = d // 2
    hw = half // 16

    def body(idx_hbm, we_hbm, wa_hbm, wv_hbm, qt1_hbm,
             rt_hbm, out_hbm, we_v, wa_v, wv_v, qt1_v, rt_v, wea_v, ixa_v,
             ixb_v, o0_v, o1_v, sem0, sem1, isema, isemb):
        wid = lax.axis_index("s") * nc + lax.axis_index("c")

        pltpu.sync_copy(we_hbm, we_v)
        pltpu.sync_copy(wa_hbm, wa_v)
        pltpu.sync_copy(wv_hbm, wv_v)
        pltpu.sync_copy(qt1_hbm, qt1_v)
        pltpu.sync_copy(rt_hbm, rt_v)

        lanes = lax.iota(jnp.int32, 16)
        segs = [lanes + 16 * k for k in range(hw)]

        def build_ea(ea):
            web = lax.shift_right_logical(ea, 4) * half
            wab = lax.bitwise_and(ea, 15) * half
            ob = ea * half
            for k in range(hw):
                we = plsc.bitcast(we_v[pl.ds(web + 16 * k, 16)], jnp.bfloat16)
                wa = plsc.bitcast(wa_v[pl.ds(wab + 16 * k, 16)], jnp.bfloat16)
                wea_v[pl.ds(ob + 16 * k, 16)] = plsc.bitcast(we + wa,
                                                             jnp.int32)

        plsc.parallel_loop(0, 512, unroll=2)(build_ea)

        hc = _C // 2
        wbase = wid * n_per_w

        def compute_chunk(base, idx_v, not_first):
            def half_body(h, buf, sem):
                def drain():
                    pltpu.make_async_copy(
                        out_hbm.at[pl.ds(0, hc * d)], buf, sem).wait()

                if not_first is True:
                    drain()
                else:
                    pl.when(not_first)(drain)

                def group_body(g):
                    off = h * hc + g * 16
                    e = idx_v[0, pl.ds(off, 16)]
                    a = idx_v[1, pl.ds(off, 16)]
                    v = idx_v[2, pl.ds(off, 16)]
                    t = idx_v[3, pl.ds(off, 16)]
                    eab = (e * 16 + a) * half
                    vb = v * half
                    qb = lax.shift_right_logical(t, 6) * half
                    rb = lax.bitwise_and(t, 63) * half

                    def tok_body(l):
                        idx = jnp.full((16,), 0, jnp.int32) + l

                        def splat(x):
                            return jnp.take_along_axis(
                                x, idx, axis=0,
                                mode="promise_in_bounds") + lanes

                        eabs = splat(eab)
                        vbs = splat(vb)
                        qbs = splat(qb)
                        rbs = splat(rb)
                        obase = (g * 16 + l) * d

                        for k in range(hw):
                            o = 16 * k

                            def bf(tab, bs):
                                w = plsc.load_gather(
                                    tab.at[pl.ds(o, tab.shape[0] - o)], [bs])
                                return plsc.bitcast(w, jnp.bfloat16)

                            def unpk(x):
                                return plsc.unpack(
                                    x, format=plsc.PackFormat.INTERLEAVED,
                                    preferred_element_type=jnp.float32)

                            w0, w1 = unpk(bf(wea_v, eabs) + bf(wv_v, vbs))
                            s1, c1 = unpk(bf(qt1_v, qbs))
                            s2, c2 = unpk(bf(rt_v, rbs))
                            sin16 = w0 + s1 * c2 + c1 * s2
                            cos16 = w1 + (c1 * c2 - s1 * s2)
                            buf[pl.ds(obase + 16 * k, 16)] = sin16
                            buf[pl.ds(obase + half + 16 * k, 16)] = cos16

                    plsc.parallel_loop(0, 16, unroll=4)(tok_body)

                plsc.parallel_loop(0, hc // 16)(group_body)
                pltpu.async_copy(
                    buf, out_hbm.at[pl.ds((base + h * hc) * d, hc * d)], sem)

            half_body(0, o0_v, sem0)
            half_body(1, o1_v, sem1)

        def idx_fetch(ci, buf, sem):
            pltpu.async_copy(
                idx_hbm.at[:, pl.ds(wbase + ci * _C, _C)], buf, sem)

        def idx_wait(buf, sem):
            pltpu.make_async_copy(
                idx_hbm.at[:, pl.ds(0, _C)], buf, sem).wait()

        idx_fetch(0, ixa_v, isema)

        def pair_body(p, carry):
            c0 = p * 2
            idx_wait(ixa_v, isema)
            idx_fetch(c0 + 1, ixb_v, isemb)
            compute_chunk(wbase + c0 * _C, ixa_v, c0 > 0)
            idx_wait(ixb_v, isemb)
            idx_fetch(c0 + 2, ixa_v, isema)
            compute_chunk(wbase + (c0 + 1) * _C, ixb_v, True)
            return carry

        lax.fori_loop(0, chunks // 2, pair_body, 0)
        idx_wait(ixa_v, isema)
        compute_chunk(wbase + (chunks - 1) * _C, ixa_v, True)
        pltpu.make_async_copy(out_hbm.at[pl.ds(0, hc * d)], o0_v, sem0).wait()
        pltpu.make_async_copy(out_hbm.at[pl.ds(0, hc * d)], o1_v, sem1).wait()

    mesh = plsc.VectorSubcoreMesh(
        core_axis_name="c", subcore_axis_name="s",
        num_cores=nc, num_subcores=ns)
    return pl.kernel(
        body,
        out_type=jax.ShapeDtypeStruct((n_tokens * d,), jnp.float32),
        mesh=mesh,
        compiler_params=pltpu.CompilerParams(needs_layout_passes=False),
        scratch_types=[
            pltpu.VMEM((32 * 64,), jnp.int32),
            pltpu.VMEM((16 * 64,), jnp.int32),
            pltpu.VMEM((32 * 64,), jnp.int32),
            pltpu.VMEM((64 * 64,), jnp.int32),
            pltpu.VMEM((64 * 64,), jnp.int32),
            pltpu.VMEM((512 * 64,), jnp.int32),
            pltpu.VMEM((4, _C), jnp.int32),
            pltpu.VMEM((4, _C), jnp.int32),
            pltpu.VMEM((_C // 2 * 128,), jnp.float32),
            pltpu.VMEM((_C // 2 * 128,), jnp.float32),
            pltpu.SemaphoreType.DMA,
            pltpu.SemaphoreType.DMA,
            pltpu.SemaphoreType.DMA,
            pltpu.SemaphoreType.DMA,
        ],
    )


def kernel(entity, attribute, value_binned, time, W_entity, W_attribute, W_value_binned):
    B, S = entity.shape
    D = W_entity.shape[1]
    half = D // 2
    N = B * S

    ratio = math.log(10000.0) / half
    f = np.exp(-ratio * np.arange(half, dtype=np.float64))
    qa = (64.0 * np.arange(64, dtype=np.float64))[:, None] * f[None, :]
    ra = np.arange(64, dtype=np.float64)[:, None] * f[None, :]
    s1, c1 = np.sin(qa), np.cos(qa)
    s2, c2 = np.sin(ra), np.cos(ra)
    qt1 = _pack_pairs_f32(s1, c1).reshape(-1)
    rt = _pack_pairs_f32(s2, c2).reshape(-1)

    def packw(w):
        return _pack_pairs_f32(w[:, :half], w[:, half:]).reshape(-1)

    idx4 = jnp.stack([entity.reshape(-1), attribute.reshape(-1),
                      value_binned.reshape(-1), time.reshape(-1)])

    call = _make_sc_call(N, D)
    out = call(
        idx4, packw(W_entity), packw(W_attribute),
        packw(W_value_binned), qt1, rt)
    return out.reshape(B, S, D)
